# baseline (speedup 1.0000x reference)
import functools

import jax
import jax.numpy as jnp
from jax import lax
from jax.experimental import pallas as pl
from jax.experimental.pallas import tpu as pltpu
from jax.experimental.pallas import tpu_sc as plsc

def _sc_gather(table, idx):
    info = plsc.get_sparse_core_info()
    nc, ns = info.num_cores, info.num_subcores
    nw = nc * ns
    B = idx.shape[0]
    V, D = table.shape
    b_per_w = B // nw
    mesh = plsc.VectorSubcoreMesh(core_axis_name="c", subcore_axis_name="s")

    @functools.partial(
        pl.kernel,
        mesh=mesh,
        compiler_params=pltpu.CompilerParams(use_tc_tiling_on_sc=False),
        out_type=jax.ShapeDtypeStruct((B, D), jnp.float32),
        scratch_types=[
            pltpu.VMEM((b_per_w,), jnp.int32),
            pltpu.VMEM((b_per_w, D), jnp.float32),
            pltpu.SemaphoreType.DMA,
        ],
    )
    def k(table_hbm, idx_hbm, out_hbm, idx_v, rows_v, sem):
        wid = lax.axis_index("s") * nc + lax.axis_index("c")
        base = wid * b_per_w
        pltpu.sync_copy(idx_hbm.at[pl.ds(base, b_per_w)], idx_v)
        pltpu.async_copy(table_hbm.at[idx_v], rows_v, sem).wait()
        pltpu.sync_copy(rows_v, out_hbm.at[pl.ds(base, b_per_w)])

    return k(table, idx)


def _fused_body(e_ref, h_ref, wir_ref, bir_ref, wiz_ref, biz_ref, win_ref,
                bin_ref, whr_ref, whz_ref, whn_ref, bhn_ref, wout_ref,
                bout_ref, logits_ref, newh_ref, h_scratch):
    i = pl.program_id(0)

    @pl.when(i == 0)
    def _():
        e = e_ref[...]
        h = h_ref[...]
        r = jax.nn.sigmoid(e @ wir_ref[...] + bir_ref[...] + h @ whr_ref[...])
        z = jax.nn.sigmoid(e @ wiz_ref[...] + biz_ref[...] + h @ whz_ref[...])
        n = jnp.tanh(e @ win_ref[...] + bin_ref[...]
                     + r * (h @ whn_ref[...] + bhn_ref[...]))
        nh = (1.0 - z) * n + z * h
        h_scratch[...] = nh.astype(jnp.bfloat16)
        newh_ref[...] = nh

    logits_ref[...] = (
        jnp.dot(h_scratch[...], wout_ref[...].astype(jnp.bfloat16),
                preferred_element_type=jnp.float32)
        + bout_ref[...])


def kernel(x, carry, embed_table, W_ir, b_ir, W_iz, b_iz, W_in, b_in,
           W_hr, W_hz, W_hn, b_hn, W_out, b_out):
    B, H = carry.shape
    V, D = embed_table.shape
    VB = 2048
    nv = pl.cdiv(V, VB)

    e = _sc_gather(embed_table, x)

    full = lambda shape: pl.BlockSpec(shape, lambda i: (0, 0))
    logits, new_h = pl.pallas_call(
        _fused_body,
        grid=(nv,),
        in_specs=[
            full((B, D)),
            full((B, H)),
            full((D, H)), full((1, H)),
            full((D, H)), full((1, H)),
            full((D, H)), full((1, H)),
            full((H, H)), full((H, H)), full((H, H)),
            full((1, H)),
            pl.BlockSpec((H, VB), lambda i: (0, i)),
            pl.BlockSpec((1, VB), lambda i: (0, i)),
        ],
        out_specs=[
            pl.BlockSpec((B, VB), lambda i: (0, i)),
            full((B, H)),
        ],
        out_shape=[
            jax.ShapeDtypeStruct((B, V), jnp.float32),
            jax.ShapeDtypeStruct((B, H), jnp.float32),
        ],
        scratch_shapes=[pltpu.VMEM((B, H), jnp.bfloat16)],
    )(
        e, carry,
        W_ir, b_ir.reshape(1, H),
        W_iz, b_iz.reshape(1, H),
        W_in, b_in.reshape(1, H),
        W_hr, W_hz, W_hn, b_hn.reshape(1, H),
        W_out, b_out.reshape(1, V),
    )
    return (logits, new_h)


# speedup vs baseline: 1.0505x; 1.0505x over previous
import functools

import jax
import jax.numpy as jnp
from jax import lax
from jax.experimental import pallas as pl
from jax.experimental.pallas import tpu as pltpu
from jax.experimental.pallas import tpu_sc as plsc

def _sc_gather(table, idx):
    info = plsc.get_sparse_core_info()
    nc, ns = info.num_cores, info.num_subcores
    nw = nc * ns
    B = idx.shape[0]
    V, D = table.shape
    b_per_w = B // nw
    mesh = plsc.VectorSubcoreMesh(core_axis_name="c", subcore_axis_name="s")

    @functools.partial(
        pl.kernel,
        mesh=mesh,
        compiler_params=pltpu.CompilerParams(use_tc_tiling_on_sc=False),
        out_type=jax.ShapeDtypeStruct((B, D), jnp.float32),
        scratch_types=[
            pltpu.VMEM((b_per_w,), jnp.int32),
            pltpu.VMEM((b_per_w, D), jnp.float32),
            pltpu.SemaphoreType.DMA,
        ],
    )
    def k(table_hbm, idx_hbm, out_hbm, idx_v, rows_v, sem):
        wid = lax.axis_index("s") * nc + lax.axis_index("c")
        base = wid * b_per_w
        pltpu.sync_copy(idx_hbm.at[pl.ds(base, b_per_w)], idx_v)
        pltpu.async_copy(table_hbm.at[idx_v], rows_v, sem).wait()
        pltpu.sync_copy(rows_v, out_hbm.at[pl.ds(base, b_per_w)])

    return k(table, idx)


def _fused_body(e_ref, h_ref, wir_ref, bir_ref, wiz_ref, biz_ref, win_ref,
                bin_ref, whr_ref, whz_ref, whn_ref, bhn_ref, wout_ref,
                bout_ref, logits_ref, newh_ref, h_scratch):
    i = pl.program_id(0)

    @pl.when(i == 0)
    def _():
        e = e_ref[...]
        h = h_ref[...]
        r = jax.nn.sigmoid(e @ wir_ref[...] + bir_ref[...] + h @ whr_ref[...])
        z = jax.nn.sigmoid(e @ wiz_ref[...] + biz_ref[...] + h @ whz_ref[...])
        n = jnp.tanh(e @ win_ref[...] + bin_ref[...]
                     + r * (h @ whn_ref[...] + bhn_ref[...]))
        nh = (1.0 - z) * n + z * h
        h_scratch[...] = nh.astype(jnp.bfloat16)
        newh_ref[...] = nh

    logits_ref[...] = (
        jnp.dot(h_scratch[...], wout_ref[...].astype(jnp.bfloat16),
                preferred_element_type=jnp.float32)
        + bout_ref[...])


def kernel(x, carry, embed_table, W_ir, b_ir, W_iz, b_iz, W_in, b_in,
           W_hr, W_hz, W_hn, b_hn, W_out, b_out):
    B, H = carry.shape
    V, D = embed_table.shape
    VB = 2048
    nv = pl.cdiv(V, VB)

    e = jnp.take(embed_table, x, axis=0)

    full = lambda shape: pl.BlockSpec(shape, lambda i: (0, 0))
    logits, new_h = pl.pallas_call(
        _fused_body,
        grid=(nv,),
        in_specs=[
            full((B, D)),
            full((B, H)),
            full((D, H)), full((1, H)),
            full((D, H)), full((1, H)),
            full((D, H)), full((1, H)),
            full((H, H)), full((H, H)), full((H, H)),
            full((1, H)),
            pl.BlockSpec((H, VB), lambda i: (0, i)),
            pl.BlockSpec((1, VB), lambda i: (0, i)),
        ],
        out_specs=[
            pl.BlockSpec((B, VB), lambda i: (0, i)),
            full((B, H)),
        ],
        out_shape=[
            jax.ShapeDtypeStruct((B, V), jnp.float32),
            jax.ShapeDtypeStruct((B, H), jnp.float32),
        ],
        scratch_shapes=[pltpu.VMEM((B, H), jnp.bfloat16)],
    )(
        e, carry,
        W_ir, b_ir.reshape(1, H),
        W_iz, b_iz.reshape(1, H),
        W_in, b_in.reshape(1, H),
        W_hr, W_hz, W_hn, b_hn.reshape(1, H),
        W_out, b_out.reshape(1, V),
    )
    return (logits, new_h)


# speedup vs baseline: 1.0572x; 1.0064x over previous
import functools

import jax
import jax.numpy as jnp
from jax import lax
from jax.experimental import pallas as pl
from jax.experimental.pallas import tpu as pltpu
from jax.experimental.pallas import tpu_sc as plsc

def _sc_gather(table, idx):
    info = plsc.get_sparse_core_info()
    nc, ns = info.num_cores, info.num_subcores
    nw = nc * ns
    B = idx.shape[0]
    V, D = table.shape
    b_per_w = B // nw
    mesh = plsc.VectorSubcoreMesh(core_axis_name="c", subcore_axis_name="s")

    @functools.partial(
        pl.kernel,
        mesh=mesh,
        compiler_params=pltpu.CompilerParams(use_tc_tiling_on_sc=False),
        out_type=jax.ShapeDtypeStruct((B, D), jnp.float32),
        scratch_types=[
            pltpu.VMEM((b_per_w,), jnp.int32),
            pltpu.VMEM((b_per_w, D), jnp.float32),
            pltpu.SemaphoreType.DMA,
        ],
    )
    def k(table_hbm, idx_hbm, out_hbm, idx_v, rows_v, sem):
        wid = lax.axis_index("s") * nc + lax.axis_index("c")
        base = wid * b_per_w
        pltpu.sync_copy(idx_hbm.at[pl.ds(base, b_per_w)], idx_v)
        pltpu.async_copy(table_hbm.at[idx_v], rows_v, sem).wait()
        pltpu.sync_copy(rows_v, out_hbm.at[pl.ds(base, b_per_w)])

    return k(table, idx)


def _fused_body(e_ref, h_ref, wir_ref, bir_ref, wiz_ref, biz_ref, win_ref,
                bin_ref, whr_ref, whz_ref, whn_ref, bhn_ref, wout_ref,
                bout_ref, logits_ref, newh_ref, h_scratch):
    i = pl.program_id(0)

    @pl.when(i == 0)
    def _():
        e = e_ref[...]
        h = h_ref[...]
        r = jax.nn.sigmoid(e @ wir_ref[...] + bir_ref[...] + h @ whr_ref[...])
        z = jax.nn.sigmoid(e @ wiz_ref[...] + biz_ref[...] + h @ whz_ref[...])
        n = jnp.tanh(e @ win_ref[...] + bin_ref[...]
                     + r * (h @ whn_ref[...] + bhn_ref[...]))
        nh = (1.0 - z) * n + z * h
        h_scratch[...] = nh.astype(jnp.bfloat16)
        newh_ref[...] = nh

    logits_ref[...] = (
        jnp.dot(h_scratch[...], wout_ref[...].astype(jnp.bfloat16),
                preferred_element_type=jnp.float32)
        + bout_ref[...])


def kernel(x, carry, embed_table, W_ir, b_ir, W_iz, b_iz, W_in, b_in,
           W_hr, W_hz, W_hn, b_hn, W_out, b_out):
    B, H = carry.shape
    V, D = embed_table.shape
    VB = 4096
    nv = pl.cdiv(V, VB)

    e = jnp.take(embed_table, x, axis=0)

    full = lambda shape: pl.BlockSpec(shape, lambda i: (0, 0))
    logits, new_h = pl.pallas_call(
        _fused_body,
        grid=(nv,),
        in_specs=[
            full((B, D)),
            full((B, H)),
            full((D, H)), full((1, H)),
            full((D, H)), full((1, H)),
            full((D, H)), full((1, H)),
            full((H, H)), full((H, H)), full((H, H)),
            full((1, H)),
            pl.BlockSpec((H, VB), lambda i: (0, i)),
            pl.BlockSpec((1, VB), lambda i: (0, i)),
        ],
        out_specs=[
            pl.BlockSpec((B, VB), lambda i: (0, i)),
            full((B, H)),
        ],
        out_shape=[
            jax.ShapeDtypeStruct((B, V), jnp.float32),
            jax.ShapeDtypeStruct((B, H), jnp.float32),
        ],
        scratch_shapes=[pltpu.VMEM((B, H), jnp.bfloat16)],
    )(
        e, carry,
        W_ir, b_ir.reshape(1, H),
        W_iz, b_iz.reshape(1, H),
        W_in, b_in.reshape(1, H),
        W_hr, W_hz, W_hn, b_hn.reshape(1, H),
        W_out, b_out.reshape(1, V),
    )
    return (logits, new_h)
